# Initial kernel scaffold; baseline (speedup 1.0000x reference)
import jax
import jax.numpy as jnp
from jax.experimental import pallas as pl


def kernel(x, edge_index, edge_attr, batch, Wm0, Wr0, We0, b0, g0, be0, Wm1, Wr1, We1, b1, g1, be1, Wm2, Wr2, We2, b2, g2, be2, RW1, Rb1, Rg, Rbe, RW2, Rb2):
    raise NotImplementedError("write your pallas kernel here")


# speedup vs baseline: 3.1899x; 3.1899x over previous
import functools

import jax
import jax.numpy as jnp
from jax import lax
from jax.experimental import pallas as pl
from jax.experimental.pallas import tpu as pltpu
from jax.experimental.pallas import tpu_sc as plsc

N = 10000
E = 320000
D_IN = 128
D_E = 16
H = 64
G = 64

NC = 2
NS = 16
NW = NC * NS
CHUNK = 128
NCHUNK = 79
E_PAD = NW * NCHUNK * CHUNK
ROWS_PER_TILE = N // NS
NEG = -1.0e30


def _q_body(ea_ref, we0_ref, we1_ref, we2_ref, q0_ref, q1_ref, q2_ref):
    i = pl.program_id(0)
    ea = ea_ref[...]
    rows = i * ea.shape[0] + lax.broadcasted_iota(jnp.int32, (ea.shape[0], 1), 0)
    valid = rows < E
    for we_ref, q_ref in ((we0_ref, q0_ref), (we1_ref, q1_ref), (we2_ref, q2_ref)):
        q = jnp.dot(ea, we_ref[...], preferred_element_type=jnp.float32)
        q_ref[...] = jnp.where(valid, q, NEG)


def _q_all(ea_p, We0, We1, We2):
    blk = 4096
    grid = E_PAD // blk
    return pl.pallas_call(
        _q_body,
        grid=(grid,),
        in_specs=[
            pl.BlockSpec((blk, D_E), lambda i: (i, 0)),
            pl.BlockSpec((D_E, H), lambda i: (0, 0)),
            pl.BlockSpec((D_E, H), lambda i: (0, 0)),
            pl.BlockSpec((D_E, H), lambda i: (0, 0)),
        ],
        out_specs=[
            pl.BlockSpec((blk, H), lambda i: (i, 0)),
            pl.BlockSpec((blk, H), lambda i: (i, 0)),
            pl.BlockSpec((blk, H), lambda i: (i, 0)),
        ],
        out_shape=[jax.ShapeDtypeStruct((E_PAD, H), jnp.float32)] * 3,
    )(ea_p, We0, We1, We2)


def _pre0_body(x_ref, wm_ref, wr_ref, p_ref, r_ref):
    x = x_ref[...]
    p_ref[...] = jnp.dot(x, wm_ref[...], preferred_element_type=jnp.float32)
    r_ref[...] = jnp.dot(x, wr_ref[...], preferred_element_type=jnp.float32)


def _pre0(x, Wm, Wr):
    return pl.pallas_call(
        _pre0_body,
        out_shape=[jax.ShapeDtypeStruct((N, H), jnp.float32)] * 2,
    )(x, Wm, Wr)


def _pre_body(r_ref, a_ref, b_ref, bias_ref, g_ref, be_ref, wm_ref, wr_ref,
              p_out, r_out):
    t = r_ref[...] + a_ref[...] + b_ref[...] + bias_ref[...]
    mu = jnp.mean(t, axis=0, keepdims=True)
    var = jnp.mean((t - mu) ** 2, axis=0, keepdims=True)
    h = jnp.maximum(g_ref[...] * (t - mu) / jnp.sqrt(var + 1e-5) + be_ref[...], 0.0)
    p_out[...] = jnp.dot(h, wm_ref[...], preferred_element_type=jnp.float32)
    r_out[...] = jnp.dot(h, wr_ref[...], preferred_element_type=jnp.float32)


def _pre(Rm, aggA, aggB, bias, g, be, Wm, Wr):
    return pl.pallas_call(
        _pre_body,
        out_shape=[jax.ShapeDtypeStruct((N, H), jnp.float32)] * 2,
    )(Rm, aggA, aggB, bias.reshape(1, H), g.reshape(1, H), be.reshape(1, H),
      Wm, Wr)


def _readout_body(r_ref, a_ref, b_ref, bias_ref, g_ref, be_ref, batch_ref,
                  rw1_ref, rb1_ref, rg_ref, rbe_ref, rw2_ref, rb2_ref, out_ref):
    t = r_ref[...] + a_ref[...] + b_ref[...] + bias_ref[...]
    mu = jnp.mean(t, axis=0, keepdims=True)
    var = jnp.mean((t - mu) ** 2, axis=0, keepdims=True)
    h = jnp.maximum(g_ref[...] * (t - mu) / jnp.sqrt(var + 1e-5) + be_ref[...], 0.0)
    gid = lax.broadcasted_iota(jnp.int32, (G, N), 0)
    onehot = (gid == batch_ref[...]).astype(jnp.float32)
    s = jnp.dot(onehot, h, preferred_element_type=jnp.float32,
                precision=lax.Precision.HIGHEST)
    cnt = jnp.sum(onehot, axis=1, keepdims=True)
    emb = s / jnp.maximum(cnt, 1.0)
    z1 = jnp.dot(emb, rw1_ref[...], preferred_element_type=jnp.float32) + rb1_ref[...]
    mu2 = jnp.mean(z1, axis=0, keepdims=True)
    var2 = jnp.mean((z1 - mu2) ** 2, axis=0, keepdims=True)
    z = jnp.maximum(rg_ref[...] * (z1 - mu2) / jnp.sqrt(var2 + 1e-5) + rbe_ref[...], 0.0)
    out_ref[...] = jnp.dot(z, rw2_ref[...], preferred_element_type=jnp.float32) + rb2_ref[...]


def _readout(Rm, aggA, aggB, bias, g, be, batch, RW1, Rb1, Rg, Rbe, RW2, Rb2):
    return pl.pallas_call(
        _readout_body,
        out_shape=jax.ShapeDtypeStruct((G, 1), jnp.float32),
    )(Rm, aggA, aggB, bias.reshape(1, H), g.reshape(1, H), be.reshape(1, H),
      batch.reshape(1, N), RW1, Rb1.reshape(1, H // 2), Rg.reshape(1, H // 2),
      Rbe.reshape(1, H // 2), RW2, Rb2.reshape(1, 1))


def _sc_body(p_hbm, q_hbm, src_hbm, dst_hbm, outa_hbm, outb_hbm,
             src_v, dst_v, rows_v, q_v, zbuf_v, agg_sh, gsem):
    cid = lax.axis_index("c")
    sid = lax.axis_index("s")
    wid = sid * NC + cid

    def _zrow(r, carry):
        for j in range(H // 16):
            zbuf_v[r, pl.ds(16 * j, 16)] = jnp.zeros((16,), jnp.float32)
        return carry
    lax.fori_loop(0, 125, _zrow, 0)
    for k in range(5):
        pltpu.sync_copy(zbuf_v, agg_sh.at[pl.ds(sid * ROWS_PER_TILE + k * 125, 125)])
    plsc.subcore_barrier()

    pltpu.sync_copy(src_hbm.at[wid], src_v)
    pltpu.sync_copy(dst_hbm.at[wid], dst_v)

    def _chunk(c, carry):
        pltpu.async_copy(p_hbm.at[src_v.at[c]], rows_v, gsem).wait()
        pltpu.sync_copy(q_hbm.at[wid, c], q_v)

        def _row(e, inner):
            for j in range(H // 16):
                s = pl.ds(16 * j, 16)
                rows_v[e, s] = jnp.maximum(rows_v[e, s] + q_v[e, s], 0.0)
            return inner
        lax.fori_loop(0, CHUNK, _row, 0)

        pltpu.sync_copy(rows_v, agg_sh.at[dst_v.at[c]], add=True)
        return carry
    lax.fori_loop(0, NCHUNK, _chunk, 0)
    plsc.subcore_barrier()

    def _drain(out):
        @pl.when(sid < NS - 1)
        def _():
            st = pl.ds(sid * 624, 624)
            pltpu.sync_copy(agg_sh.at[st], out.at[st])

        @pl.when(sid == NS - 1)
        def _():
            st = pl.ds(624 * (NS - 1), 640)
            pltpu.sync_copy(agg_sh.at[st], out.at[st])

    @pl.when(cid == 0)
    def _():
        _drain(outa_hbm)

    @pl.when(cid == 1)
    def _():
        _drain(outb_hbm)


_sc_agg = pl.kernel(
    _sc_body,
    out_type=[jax.ShapeDtypeStruct((N, H), jnp.float32)] * 2,
    mesh=plsc.VectorSubcoreMesh(core_axis_name="c", subcore_axis_name="s"),
    compiler_params=pltpu.CompilerParams(use_tc_tiling_on_sc=False),
    scratch_types=[
        pltpu.VMEM((NCHUNK, CHUNK), jnp.int32),
        pltpu.VMEM((NCHUNK, CHUNK), jnp.int32),
        pltpu.VMEM((CHUNK, H), jnp.float32),
        pltpu.VMEM((CHUNK, H), jnp.float32),
        pltpu.VMEM((125, H), jnp.float32),
        pltpu.VMEM_SHARED((N, H), jnp.float32),
        pltpu.SemaphoreType.DMA,
    ],
)


def kernel(x, edge_index, edge_attr, batch,
           Wm0, Wr0, We0, b0, g0, be0,
           Wm1, Wr1, We1, b1, g1, be1,
           Wm2, Wr2, We2, b2, g2, be2,
           RW1, Rb1, Rg, Rbe, RW2, Rb2):
    pad = E_PAD - E
    src = jnp.concatenate([edge_index[0], jnp.zeros((pad,), jnp.int32)])
    dst = jnp.concatenate([edge_index[1], jnp.zeros((pad,), jnp.int32)])
    src_p = src.reshape(NW, NCHUNK, CHUNK)
    dst_p = dst.reshape(NW, NCHUNK, CHUNK)
    ea_p = jnp.concatenate([edge_attr, jnp.zeros((pad, D_E), jnp.float32)])

    q0, q1, q2 = _q_all(ea_p, We0, We1, We2)
    qs = [q.reshape(NW, NCHUNK, CHUNK, H) for q in (q0, q1, q2)]

    P, Rm = _pre0(x, Wm0, Wr0)
    layer = ((b0, g0, be0, Wm1, Wr1), (b1, g1, be1, Wm2, Wr2))
    for l in range(3):
        aggA, aggB = _sc_agg(P, qs[l], src_p, dst_p)
        if l < 2:
            bias, g, be, Wm, Wr = layer[l]
            P, Rm = _pre(Rm, aggA, aggB, bias, g, be, Wm, Wr)

    out = _readout(Rm, aggA, aggB, b2, g2, be2, batch,
                   RW1, Rb1, Rg, Rbe, RW2, Rb2)
    return out[:, 0]


# speedup vs baseline: 4.0674x; 1.2751x over previous
import functools

import jax
import jax.numpy as jnp
from jax import lax
from jax.experimental import pallas as pl
from jax.experimental.pallas import tpu as pltpu
from jax.experimental.pallas import tpu_sc as plsc

N = 10000
E = 320000
D_IN = 128
D_E = 16
H = 64
G = 64

NC = 2
NS = 16
NW = NC * NS
CHUNK = 128
NCHUNK = 79
E_PAD = NW * NCHUNK * CHUNK
ROWS_PER_TILE = N // NS
NEG = -1.0e30


def _q_body(ea_ref, we0_ref, we1_ref, we2_ref, q0_ref, q1_ref, q2_ref):
    i = pl.program_id(0)
    ea = ea_ref[...]
    rows = i * ea.shape[0] + lax.broadcasted_iota(jnp.int32, (ea.shape[0], 1), 0)
    valid = rows < E
    for we_ref, q_ref in ((we0_ref, q0_ref), (we1_ref, q1_ref), (we2_ref, q2_ref)):
        q = jnp.dot(ea, we_ref[...], preferred_element_type=jnp.float32)
        q_ref[...] = jnp.where(valid, q, NEG)


def _q_all(ea_p, We0, We1, We2):
    blk = 4096
    grid = E_PAD // blk
    return pl.pallas_call(
        _q_body,
        grid=(grid,),
        in_specs=[
            pl.BlockSpec((blk, D_E), lambda i: (i, 0)),
            pl.BlockSpec((D_E, H), lambda i: (0, 0)),
            pl.BlockSpec((D_E, H), lambda i: (0, 0)),
            pl.BlockSpec((D_E, H), lambda i: (0, 0)),
        ],
        out_specs=[
            pl.BlockSpec((blk, H), lambda i: (i, 0)),
            pl.BlockSpec((blk, H), lambda i: (i, 0)),
            pl.BlockSpec((blk, H), lambda i: (i, 0)),
        ],
        out_shape=[jax.ShapeDtypeStruct((E_PAD, H), jnp.float32)] * 3,
    )(ea_p, We0, We1, We2)


def _pre0_body(x_ref, wm_ref, wr_ref, p_ref, r_ref):
    x = x_ref[...]
    p_ref[...] = jnp.dot(x, wm_ref[...], preferred_element_type=jnp.float32)
    r_ref[...] = jnp.dot(x, wr_ref[...], preferred_element_type=jnp.float32)


def _pre0(x, Wm, Wr):
    return pl.pallas_call(
        _pre0_body,
        out_shape=[jax.ShapeDtypeStruct((N, H), jnp.float32)] * 2,
    )(x, Wm, Wr)


def _pre_body(r_ref, a_ref, b_ref, bias_ref, g_ref, be_ref, wm_ref, wr_ref,
              p_out, r_out):
    t = r_ref[...] + a_ref[...] + b_ref[...] + bias_ref[...]
    mu = jnp.mean(t, axis=0, keepdims=True)
    var = jnp.mean((t - mu) ** 2, axis=0, keepdims=True)
    h = jnp.maximum(g_ref[...] * (t - mu) / jnp.sqrt(var + 1e-5) + be_ref[...], 0.0)
    p_out[...] = jnp.dot(h, wm_ref[...], preferred_element_type=jnp.float32)
    r_out[...] = jnp.dot(h, wr_ref[...], preferred_element_type=jnp.float32)


def _pre(Rm, aggA, aggB, bias, g, be, Wm, Wr):
    return pl.pallas_call(
        _pre_body,
        out_shape=[jax.ShapeDtypeStruct((N, H), jnp.float32)] * 2,
    )(Rm, aggA, aggB, bias.reshape(1, H), g.reshape(1, H), be.reshape(1, H),
      Wm, Wr)


def _readout_body(r_ref, a_ref, b_ref, bias_ref, g_ref, be_ref, batch_ref,
                  rw1_ref, rb1_ref, rg_ref, rbe_ref, rw2_ref, rb2_ref, out_ref):
    t = r_ref[...] + a_ref[...] + b_ref[...] + bias_ref[...]
    mu = jnp.mean(t, axis=0, keepdims=True)
    var = jnp.mean((t - mu) ** 2, axis=0, keepdims=True)
    h = jnp.maximum(g_ref[...] * (t - mu) / jnp.sqrt(var + 1e-5) + be_ref[...], 0.0)
    gid = lax.broadcasted_iota(jnp.int32, (G, N), 0)
    onehot = (gid == batch_ref[...]).astype(jnp.float32)
    s = jnp.dot(onehot, h, preferred_element_type=jnp.float32,
                precision=lax.Precision.HIGHEST)
    cnt = jnp.sum(onehot, axis=1, keepdims=True)
    emb = s / jnp.maximum(cnt, 1.0)
    z1 = jnp.dot(emb, rw1_ref[...], preferred_element_type=jnp.float32) + rb1_ref[...]
    mu2 = jnp.mean(z1, axis=0, keepdims=True)
    var2 = jnp.mean((z1 - mu2) ** 2, axis=0, keepdims=True)
    z = jnp.maximum(rg_ref[...] * (z1 - mu2) / jnp.sqrt(var2 + 1e-5) + rbe_ref[...], 0.0)
    out_ref[...] = jnp.dot(z, rw2_ref[...], preferred_element_type=jnp.float32) + rb2_ref[...]


def _readout(Rm, aggA, aggB, bias, g, be, batch, RW1, Rb1, Rg, Rbe, RW2, Rb2):
    return pl.pallas_call(
        _readout_body,
        out_shape=jax.ShapeDtypeStruct((G, 1), jnp.float32),
    )(Rm, aggA, aggB, bias.reshape(1, H), g.reshape(1, H), be.reshape(1, H),
      batch.reshape(1, N), RW1, Rb1.reshape(1, H // 2), Rg.reshape(1, H // 2),
      Rbe.reshape(1, H // 2), RW2, Rb2.reshape(1, 1))


def _sc_body(p_hbm, q_hbm, src_hbm, dst_hbm, outa_hbm, outb_hbm,
             src_v, dst_v, rows_v, q_v, zbuf_v, agg_sh, p_sh, gsem):
    cid = lax.axis_index("c")
    sid = lax.axis_index("s")
    wid = sid * NC + cid

    @pl.when(sid < NS - 1)
    def _():
        st = pl.ds(sid * 624, 624)
        pltpu.sync_copy(p_hbm.at[st], p_sh.at[st])

    @pl.when(sid == NS - 1)
    def _():
        st = pl.ds(624 * (NS - 1), 640)
        pltpu.sync_copy(p_hbm.at[st], p_sh.at[st])

    def _zrow(r, carry):
        for j in range(H // 16):
            zbuf_v[r, pl.ds(16 * j, 16)] = jnp.zeros((16,), jnp.float32)
        return carry
    lax.fori_loop(0, 125, _zrow, 0)
    for k in range(5):
        pltpu.sync_copy(zbuf_v, agg_sh.at[pl.ds(sid * ROWS_PER_TILE + k * 125, 125)])
    plsc.subcore_barrier()

    pltpu.sync_copy(src_hbm.at[wid], src_v)
    pltpu.sync_copy(dst_hbm.at[wid], dst_v)

    def _chunk(c, carry):
        pltpu.async_copy(p_sh.at[src_v.at[c]], rows_v, gsem).wait()
        pltpu.sync_copy(q_hbm.at[wid, c], q_v)

        def _row(e, inner):
            for j in range(H // 16):
                s = pl.ds(16 * j, 16)
                rows_v[e, s] = jnp.maximum(rows_v[e, s] + q_v[e, s], 0.0)
            return inner
        lax.fori_loop(0, CHUNK, _row, 0)

        pltpu.sync_copy(rows_v, agg_sh.at[dst_v.at[c]], add=True)
        return carry
    lax.fori_loop(0, NCHUNK, _chunk, 0)
    plsc.subcore_barrier()

    def _drain(out):
        @pl.when(sid < NS - 1)
        def _():
            st = pl.ds(sid * 624, 624)
            pltpu.sync_copy(agg_sh.at[st], out.at[st])

        @pl.when(sid == NS - 1)
        def _():
            st = pl.ds(624 * (NS - 1), 640)
            pltpu.sync_copy(agg_sh.at[st], out.at[st])

    @pl.when(cid == 0)
    def _():
        _drain(outa_hbm)

    @pl.when(cid == 1)
    def _():
        _drain(outb_hbm)


_sc_agg = pl.kernel(
    _sc_body,
    out_type=[jax.ShapeDtypeStruct((N, H), jnp.float32)] * 2,
    mesh=plsc.VectorSubcoreMesh(core_axis_name="c", subcore_axis_name="s"),
    compiler_params=pltpu.CompilerParams(use_tc_tiling_on_sc=False),
    scratch_types=[
        pltpu.VMEM((NCHUNK, CHUNK), jnp.int32),
        pltpu.VMEM((NCHUNK, CHUNK), jnp.int32),
        pltpu.VMEM((CHUNK, H), jnp.float32),
        pltpu.VMEM((CHUNK, H), jnp.float32),
        pltpu.VMEM((125, H), jnp.float32),
        pltpu.VMEM_SHARED((N, H), jnp.float32),
        pltpu.VMEM_SHARED((N, H), jnp.float32),
        pltpu.SemaphoreType.DMA,
    ],
)


def kernel(x, edge_index, edge_attr, batch,
           Wm0, Wr0, We0, b0, g0, be0,
           Wm1, Wr1, We1, b1, g1, be1,
           Wm2, Wr2, We2, b2, g2, be2,
           RW1, Rb1, Rg, Rbe, RW2, Rb2):
    pad = E_PAD - E
    src = jnp.concatenate([edge_index[0], jnp.zeros((pad,), jnp.int32)])
    dst = jnp.concatenate([edge_index[1], jnp.zeros((pad,), jnp.int32)])
    src_p = src.reshape(NW, NCHUNK, CHUNK)
    dst_p = dst.reshape(NW, NCHUNK, CHUNK)
    ea_p = jnp.concatenate([edge_attr, jnp.zeros((pad, D_E), jnp.float32)])

    q0, q1, q2 = _q_all(ea_p, We0, We1, We2)
    qs = [q.reshape(NW, NCHUNK, CHUNK, H) for q in (q0, q1, q2)]

    P, Rm = _pre0(x, Wm0, Wr0)
    layer = ((b0, g0, be0, Wm1, Wr1), (b1, g1, be1, Wm2, Wr2))
    for l in range(3):
        aggA, aggB = _sc_agg(P, qs[l], src_p, dst_p)
        if l < 2:
            bias, g, be, Wm, Wr = layer[l]
            P, Rm = _pre(Rm, aggA, aggB, bias, g, be, Wm, Wr)

    out = _readout(Rm, aggA, aggB, b2, g2, be2, batch,
                   RW1, Rb1, Rg, Rbe, RW2, Rb2)
    return out[:, 0]


# speedup vs baseline: 4.3703x; 1.0745x over previous
import functools

import jax
import jax.numpy as jnp
from jax import lax
from jax.experimental import pallas as pl
from jax.experimental.pallas import tpu as pltpu
from jax.experimental.pallas import tpu_sc as plsc

N = 10000
E = 320000
D_IN = 128
D_E = 16
H = 64
G = 64

NC = 2
NS = 16
NW = NC * NS
CHUNK = 64
NCHUNK = 160
NPAIR = NCHUNK // 2
E_PAD = NW * NCHUNK * CHUNK
ROWS_PER_TILE = N // NS
NEG = -1.0e30


def _q_body(ea_ref, we_ref, q_ref):
    i = pl.program_id(0)
    ea = ea_ref[...]
    rows = i * ea.shape[0] + lax.broadcasted_iota(jnp.int32, (ea.shape[0], 1), 0)
    valid = rows < E
    q = jnp.dot(ea, we_ref[...], preferred_element_type=jnp.float32)
    q_ref[...] = jnp.where(valid, q, NEG)


def _q_one(ea_p, We):
    blk = 4096
    grid = E_PAD // blk
    return pl.pallas_call(
        _q_body,
        grid=(grid,),
        in_specs=[
            pl.BlockSpec((blk, D_E), lambda i: (i, 0)),
            pl.BlockSpec((D_E, H), lambda i: (0, 0)),
        ],
        out_specs=pl.BlockSpec((blk, H), lambda i: (i, 0)),
        out_shape=jax.ShapeDtypeStruct((E_PAD, H), jnp.float32),
    )(ea_p, We)


def _pre0_body(x_ref, wm_ref, wr_ref, p_ref, r_ref):
    x = x_ref[...]
    p_ref[...] = jnp.dot(x, wm_ref[...], preferred_element_type=jnp.float32)
    r_ref[...] = jnp.dot(x, wr_ref[...], preferred_element_type=jnp.float32)


def _pre0(x, Wm, Wr):
    return pl.pallas_call(
        _pre0_body,
        out_shape=[jax.ShapeDtypeStruct((N, H), jnp.float32)] * 2,
    )(x, Wm, Wr)


def _pre_body(r_ref, a_ref, b_ref, bias_ref, g_ref, be_ref, wm_ref, wr_ref,
              p_out, r_out):
    t = r_ref[...] + a_ref[...] + b_ref[...] + bias_ref[...]
    mu = jnp.mean(t, axis=0, keepdims=True)
    var = jnp.mean((t - mu) ** 2, axis=0, keepdims=True)
    h = jnp.maximum(g_ref[...] * (t - mu) / jnp.sqrt(var + 1e-5) + be_ref[...], 0.0)
    p_out[...] = jnp.dot(h, wm_ref[...], preferred_element_type=jnp.float32)
    r_out[...] = jnp.dot(h, wr_ref[...], preferred_element_type=jnp.float32)


def _pre(Rm, aggA, aggB, bias, g, be, Wm, Wr):
    return pl.pallas_call(
        _pre_body,
        out_shape=[jax.ShapeDtypeStruct((N, H), jnp.float32)] * 2,
    )(Rm, aggA, aggB, bias.reshape(1, H), g.reshape(1, H), be.reshape(1, H),
      Wm, Wr)


def _readout_body(r_ref, a_ref, b_ref, bias_ref, g_ref, be_ref, batch_ref,
                  rw1_ref, rb1_ref, rg_ref, rbe_ref, rw2_ref, rb2_ref, out_ref):
    t = r_ref[...] + a_ref[...] + b_ref[...] + bias_ref[...]
    mu = jnp.mean(t, axis=0, keepdims=True)
    var = jnp.mean((t - mu) ** 2, axis=0, keepdims=True)
    h = jnp.maximum(g_ref[...] * (t - mu) / jnp.sqrt(var + 1e-5) + be_ref[...], 0.0)
    gid = lax.broadcasted_iota(jnp.int32, (G, N), 0)
    onehot = (gid == batch_ref[...]).astype(jnp.float32)
    s = jnp.dot(onehot, h, preferred_element_type=jnp.float32,
                precision=lax.Precision.HIGHEST)
    cnt = jnp.sum(onehot, axis=1, keepdims=True)
    emb = s / jnp.maximum(cnt, 1.0)
    z1 = jnp.dot(emb, rw1_ref[...], preferred_element_type=jnp.float32) + rb1_ref[...]
    mu2 = jnp.mean(z1, axis=0, keepdims=True)
    var2 = jnp.mean((z1 - mu2) ** 2, axis=0, keepdims=True)
    z = jnp.maximum(rg_ref[...] * (z1 - mu2) / jnp.sqrt(var2 + 1e-5) + rbe_ref[...], 0.0)
    out_ref[...] = jnp.dot(z, rw2_ref[...], preferred_element_type=jnp.float32) + rb2_ref[...]


def _readout(Rm, aggA, aggB, bias, g, be, batch, RW1, Rb1, Rg, Rbe, RW2, Rb2):
    return pl.pallas_call(
        _readout_body,
        out_shape=jax.ShapeDtypeStruct((G, 1), jnp.float32),
    )(Rm, aggA, aggB, bias.reshape(1, H), g.reshape(1, H), be.reshape(1, H),
      batch.reshape(1, N), RW1, Rb1.reshape(1, H // 2), Rg.reshape(1, H // 2),
      Rbe.reshape(1, H // 2), RW2, Rb2.reshape(1, 1))


def _sc_body(p_hbm, q_hbm, src_hbm, dst_hbm, outa_hbm, outb_hbm,
             src_v, dst_v, rows_a, rows_b, q_a, q_b, zbuf_v, agg_sh, p_sh,
             gsa, gsb, qsa, qsb):
    cid = lax.axis_index("c")
    sid = lax.axis_index("s")
    wid = sid * NC + cid

    @pl.when(sid < NS - 1)
    def _():
        st = pl.ds(sid * 624, 624)
        pltpu.sync_copy(p_hbm.at[st], p_sh.at[st])

    @pl.when(sid == NS - 1)
    def _():
        st = pl.ds(624 * (NS - 1), 640)
        pltpu.sync_copy(p_hbm.at[st], p_sh.at[st])

    def _zrow(r, carry):
        for j in range(H // 16):
            zbuf_v[r, pl.ds(16 * j, 16)] = jnp.zeros((16,), jnp.float32)
        return carry
    lax.fori_loop(0, 125, _zrow, 0)
    for k in range(5):
        pltpu.sync_copy(zbuf_v, agg_sh.at[pl.ds(sid * ROWS_PER_TILE + k * 125, 125)])
    plsc.subcore_barrier()

    pltpu.sync_copy(src_hbm.at[wid], src_v)
    pltpu.sync_copy(dst_hbm.at[wid], dst_v)

    def _relu_add(rows, q):
        def _row(e, inner):
            for j in range(H // 16):
                s = pl.ds(16 * j, 16)
                rows[e, s] = jnp.maximum(rows[e, s] + q[e, s], 0.0)
            return inner
        lax.fori_loop(0, CHUNK, _row, 0)

    def _start(c, rows, q, gs, qs):
        pltpu.async_copy(p_sh.at[src_v.at[c]], rows, gs)
        pltpu.async_copy(q_hbm.at[wid, c], q, qs)

    def _wait(c, rows, q, gs, qs):
        pltpu.make_async_copy(p_sh.at[src_v.at[c]], rows, gs).wait()
        pltpu.make_async_copy(q_hbm.at[wid, c], q, qs).wait()

    _start(0, rows_a, q_a, gsa, qsa)

    def _pair(c2, carry):
        c0 = 2 * c2
        c1 = c0 + 1
        _wait(c0, rows_a, q_a, gsa, qsa)
        _start(c1, rows_b, q_b, gsb, qsb)
        _relu_add(rows_a, q_a)
        pltpu.sync_copy(rows_a, agg_sh.at[dst_v.at[c0]], add=True)
        _wait(c1, rows_b, q_b, gsb, qsb)

        @pl.when(c2 + 1 < NPAIR)
        def _():
            _start(c0 + 2, rows_a, q_a, gsa, qsa)
        _relu_add(rows_b, q_b)
        pltpu.sync_copy(rows_b, agg_sh.at[dst_v.at[c1]], add=True)
        return carry
    lax.fori_loop(0, NPAIR, _pair, 0)
    plsc.subcore_barrier()

    def _drain(out):
        @pl.when(sid < NS - 1)
        def _():
            st = pl.ds(sid * 624, 624)
            pltpu.sync_copy(agg_sh.at[st], out.at[st])

        @pl.when(sid == NS - 1)
        def _():
            st = pl.ds(624 * (NS - 1), 640)
            pltpu.sync_copy(agg_sh.at[st], out.at[st])

    @pl.when(cid == 0)
    def _():
        _drain(outa_hbm)

    @pl.when(cid == 1)
    def _():
        _drain(outb_hbm)


_sc_agg = pl.kernel(
    _sc_body,
    out_type=[jax.ShapeDtypeStruct((N, H), jnp.float32)] * 2,
    mesh=plsc.VectorSubcoreMesh(core_axis_name="c", subcore_axis_name="s"),
    compiler_params=pltpu.CompilerParams(use_tc_tiling_on_sc=False),
    scratch_types=[
        pltpu.VMEM((NCHUNK, CHUNK), jnp.int32),
        pltpu.VMEM((NCHUNK, CHUNK), jnp.int32),
        pltpu.VMEM((CHUNK, H), jnp.float32),
        pltpu.VMEM((CHUNK, H), jnp.float32),
        pltpu.VMEM((CHUNK, H), jnp.float32),
        pltpu.VMEM((CHUNK, H), jnp.float32),
        pltpu.VMEM((125, H), jnp.float32),
        pltpu.VMEM_SHARED((N, H), jnp.float32),
        pltpu.VMEM_SHARED((N, H), jnp.float32),
        pltpu.SemaphoreType.DMA,
        pltpu.SemaphoreType.DMA,
        pltpu.SemaphoreType.DMA,
        pltpu.SemaphoreType.DMA,
    ],
)


def kernel(x, edge_index, edge_attr, batch,
           Wm0, Wr0, We0, b0, g0, be0,
           Wm1, Wr1, We1, b1, g1, be1,
           Wm2, Wr2, We2, b2, g2, be2,
           RW1, Rb1, Rg, Rbe, RW2, Rb2):
    pad = E_PAD - E
    src = jnp.concatenate([edge_index[0], jnp.zeros((pad,), jnp.int32)])
    dst = jnp.concatenate([edge_index[1], jnp.zeros((pad,), jnp.int32)])
    src_p = src.reshape(NW, NCHUNK, CHUNK)
    dst_p = dst.reshape(NW, NCHUNK, CHUNK)
    ea_p = jnp.concatenate([edge_attr, jnp.zeros((pad, D_E), jnp.float32)])

    qs = [_q_one(ea_p, We).reshape(NW, NCHUNK, CHUNK, H)
          for We in (We0, We1, We2)]

    P, Rm = _pre0(x, Wm0, Wr0)
    layer = ((b0, g0, be0, Wm1, Wr1), (b1, g1, be1, Wm2, Wr2))
    for l in range(3):
        aggA, aggB = _sc_agg(P, qs[l], src_p, dst_p)
        if l < 2:
            bias, g, be, Wm, Wr = layer[l]
            P, Rm = _pre(Rm, aggA, aggB, bias, g, be, Wm, Wr)

    out = _readout(Rm, aggA, aggB, b2, g2, be2, batch,
                   RW1, Rb1, Rg, Rbe, RW2, Rb2)
    return out[:, 0]


# speedup vs baseline: 4.7317x; 1.0827x over previous
import functools

import jax
import jax.numpy as jnp
from jax import lax
from jax.experimental import pallas as pl
from jax.experimental.pallas import tpu as pltpu
from jax.experimental.pallas import tpu_sc as plsc

N = 10000
E = 320000
D_IN = 128
D_E = 16
H = 64
G = 64

NC = 2
NS = 16
NW = NC * NS
CHUNK = 64
NCHUNK = 160
NPAIR = NCHUNK // 2
E_PAD = NW * NCHUNK * CHUNK
ROWS_PER_TILE = N // NS
NEG = -1.0e30


def _q_body(ea_ref, we0_ref, we1_ref, we2_ref, q0_ref, q1_ref, q2_ref):
    i = pl.program_id(0)
    ea = ea_ref[...]
    rows = i * ea.shape[0] + lax.broadcasted_iota(jnp.int32, (ea.shape[0], 1), 0)
    valid = rows < E
    for we_ref, q_ref in ((we0_ref, q0_ref), (we1_ref, q1_ref), (we2_ref, q2_ref)):
        q = jnp.dot(ea, we_ref[...], preferred_element_type=jnp.float32)
        q_ref[...] = jnp.where(valid, q, NEG)


def _q_all(ea_p, We0, We1, We2):
    blk = 4096
    grid = E_PAD // blk
    return pl.pallas_call(
        _q_body,
        grid=(grid,),
        in_specs=[
            pl.BlockSpec((blk, D_E), lambda i: (i, 0)),
            pl.BlockSpec((D_E, H), lambda i: (0, 0)),
            pl.BlockSpec((D_E, H), lambda i: (0, 0)),
            pl.BlockSpec((D_E, H), lambda i: (0, 0)),
        ],
        out_specs=[
            pl.BlockSpec((blk, H), lambda i: (i, 0)),
            pl.BlockSpec((blk, H), lambda i: (i, 0)),
            pl.BlockSpec((blk, H), lambda i: (i, 0)),
        ],
        out_shape=[jax.ShapeDtypeStruct((E_PAD, H), jnp.float32)] * 3,
    )(ea_p, We0, We1, We2)


def _pre0_body(x_ref, wm_ref, wr_ref, p_ref, r_ref):
    x = x_ref[...]
    p_ref[...] = jnp.dot(x, wm_ref[...], preferred_element_type=jnp.float32)
    r_ref[...] = jnp.dot(x, wr_ref[...], preferred_element_type=jnp.float32)


def _pre0(x, Wm, Wr):
    return pl.pallas_call(
        _pre0_body,
        out_shape=[jax.ShapeDtypeStruct((N, H), jnp.float32)] * 2,
    )(x, Wm, Wr)


def _pre_body(r_ref, a_ref, b_ref, bias_ref, g_ref, be_ref, wm_ref, wr_ref,
              p_out, r_out):
    t = r_ref[...] + a_ref[...] + b_ref[...] + bias_ref[...]
    mu = jnp.mean(t, axis=0, keepdims=True)
    var = jnp.mean((t - mu) ** 2, axis=0, keepdims=True)
    h = jnp.maximum(g_ref[...] * (t - mu) / jnp.sqrt(var + 1e-5) + be_ref[...], 0.0)
    p_out[...] = jnp.dot(h, wm_ref[...], preferred_element_type=jnp.float32)
    r_out[...] = jnp.dot(h, wr_ref[...], preferred_element_type=jnp.float32)


def _pre(Rm, aggA, aggB, bias, g, be, Wm, Wr):
    return pl.pallas_call(
        _pre_body,
        out_shape=[jax.ShapeDtypeStruct((N, H), jnp.float32)] * 2,
    )(Rm, aggA, aggB, bias.reshape(1, H), g.reshape(1, H), be.reshape(1, H),
      Wm, Wr)


def _readout_body(r_ref, a_ref, b_ref, bias_ref, g_ref, be_ref, batch_ref,
                  rw1_ref, rb1_ref, rg_ref, rbe_ref, rw2_ref, rb2_ref, out_ref):
    t = r_ref[...] + a_ref[...] + b_ref[...] + bias_ref[...]
    mu = jnp.mean(t, axis=0, keepdims=True)
    var = jnp.mean((t - mu) ** 2, axis=0, keepdims=True)
    h = jnp.maximum(g_ref[...] * (t - mu) / jnp.sqrt(var + 1e-5) + be_ref[...], 0.0)
    gid = lax.broadcasted_iota(jnp.int32, (G, N), 0)
    onehot = (gid == batch_ref[...]).astype(jnp.float32)
    s = jnp.dot(onehot, h, preferred_element_type=jnp.float32,
                precision=lax.Precision.HIGHEST)
    cnt = jnp.sum(onehot, axis=1, keepdims=True)
    emb = s / jnp.maximum(cnt, 1.0)
    z1 = jnp.dot(emb, rw1_ref[...], preferred_element_type=jnp.float32) + rb1_ref[...]
    mu2 = jnp.mean(z1, axis=0, keepdims=True)
    var2 = jnp.mean((z1 - mu2) ** 2, axis=0, keepdims=True)
    z = jnp.maximum(rg_ref[...] * (z1 - mu2) / jnp.sqrt(var2 + 1e-5) + rbe_ref[...], 0.0)
    out_ref[...] = jnp.dot(z, rw2_ref[...], preferred_element_type=jnp.float32) + rb2_ref[...]


def _readout(Rm, aggA, aggB, bias, g, be, batch, RW1, Rb1, Rg, Rbe, RW2, Rb2):
    return pl.pallas_call(
        _readout_body,
        out_shape=jax.ShapeDtypeStruct((G, 1), jnp.float32),
    )(Rm, aggA, aggB, bias.reshape(1, H), g.reshape(1, H), be.reshape(1, H),
      batch.reshape(1, N), RW1, Rb1.reshape(1, H // 2), Rg.reshape(1, H // 2),
      Rbe.reshape(1, H // 2), RW2, Rb2.reshape(1, 1))


def _sc_body(p_hbm, q_hbm, src_hbm, dst_hbm, outa_hbm, outb_hbm,
             src_v, dst_v, rows_a, rows_b, q_a, q_b, zbuf_v, agg_sh, p_sh,
             gsa, gsb, qsa, qsb):
    cid = lax.axis_index("c")
    sid = lax.axis_index("s")
    wid = sid * NC + cid

    @pl.when(sid < NS - 1)
    def _():
        st = pl.ds(sid * 624, 624)
        pltpu.sync_copy(p_hbm.at[st], p_sh.at[st])

    @pl.when(sid == NS - 1)
    def _():
        st = pl.ds(624 * (NS - 1), 640)
        pltpu.sync_copy(p_hbm.at[st], p_sh.at[st])

    def _zrow(r, carry):
        for j in range(H // 16):
            zbuf_v[r, pl.ds(16 * j, 16)] = jnp.zeros((16,), jnp.float32)
        return carry
    lax.fori_loop(0, 125, _zrow, 0)
    for k in range(5):
        pltpu.sync_copy(zbuf_v, agg_sh.at[pl.ds(sid * ROWS_PER_TILE + k * 125, 125)])
    plsc.subcore_barrier()

    pltpu.sync_copy(src_hbm.at[wid], src_v)
    pltpu.sync_copy(dst_hbm.at[wid], dst_v)

    def _relu_add(rows, q):
        def _row(e, inner):
            for j in range(H // 16):
                s = pl.ds(16 * j, 16)
                rows[e, s] = jnp.maximum(rows[e, s] + q[e, s], 0.0)
            return inner
        lax.fori_loop(0, CHUNK, _row, 0)

    def _start(c, rows, q, gs, qs):
        pltpu.async_copy(p_sh.at[src_v.at[c]], rows, gs)
        pltpu.async_copy(q_hbm.at[wid, c], q, qs)

    def _wait(c, rows, q, gs, qs):
        pltpu.make_async_copy(p_sh.at[src_v.at[c]], rows, gs).wait()
        pltpu.make_async_copy(q_hbm.at[wid, c], q, qs).wait()

    _start(0, rows_a, q_a, gsa, qsa)

    def _pair(c2, carry):
        c0 = 2 * c2
        c1 = c0 + 1
        _wait(c0, rows_a, q_a, gsa, qsa)
        _start(c1, rows_b, q_b, gsb, qsb)
        _relu_add(rows_a, q_a)
        pltpu.sync_copy(rows_a, agg_sh.at[dst_v.at[c0]], add=True)
        _wait(c1, rows_b, q_b, gsb, qsb)

        @pl.when(c2 + 1 < NPAIR)
        def _():
            _start(c0 + 2, rows_a, q_a, gsa, qsa)
        _relu_add(rows_b, q_b)
        pltpu.sync_copy(rows_b, agg_sh.at[dst_v.at[c1]], add=True)
        return carry
    lax.fori_loop(0, NPAIR, _pair, 0)
    plsc.subcore_barrier()

    def _drain(out):
        @pl.when(sid < NS - 1)
        def _():
            st = pl.ds(sid * 624, 624)
            pltpu.sync_copy(agg_sh.at[st], out.at[st])

        @pl.when(sid == NS - 1)
        def _():
            st = pl.ds(624 * (NS - 1), 640)
            pltpu.sync_copy(agg_sh.at[st], out.at[st])

    @pl.when(cid == 0)
    def _():
        _drain(outa_hbm)

    @pl.when(cid == 1)
    def _():
        _drain(outb_hbm)


_sc_agg = pl.kernel(
    _sc_body,
    out_type=[jax.ShapeDtypeStruct((N, H), jnp.float32)] * 2,
    mesh=plsc.VectorSubcoreMesh(core_axis_name="c", subcore_axis_name="s"),
    compiler_params=pltpu.CompilerParams(use_tc_tiling_on_sc=False),
    scratch_types=[
        pltpu.VMEM((NCHUNK, CHUNK), jnp.int32),
        pltpu.VMEM((NCHUNK, CHUNK), jnp.int32),
        pltpu.VMEM((CHUNK, H), jnp.float32),
        pltpu.VMEM((CHUNK, H), jnp.float32),
        pltpu.VMEM((CHUNK, H), jnp.float32),
        pltpu.VMEM((CHUNK, H), jnp.float32),
        pltpu.VMEM((125, H), jnp.float32),
        pltpu.VMEM_SHARED((N, H), jnp.float32),
        pltpu.VMEM_SHARED((N, H), jnp.float32),
        pltpu.SemaphoreType.DMA,
        pltpu.SemaphoreType.DMA,
        pltpu.SemaphoreType.DMA,
        pltpu.SemaphoreType.DMA,
    ],
)


def kernel(x, edge_index, edge_attr, batch,
           Wm0, Wr0, We0, b0, g0, be0,
           Wm1, Wr1, We1, b1, g1, be1,
           Wm2, Wr2, We2, b2, g2, be2,
           RW1, Rb1, Rg, Rbe, RW2, Rb2):
    pad = E_PAD - E
    src = jnp.concatenate([edge_index[0], jnp.zeros((pad,), jnp.int32)])
    dst = jnp.concatenate([edge_index[1], jnp.zeros((pad,), jnp.int32)])
    src_p = src.reshape(NW, NCHUNK, CHUNK)
    dst_p = dst.reshape(NW, NCHUNK, CHUNK)
    ea_p = jnp.concatenate([edge_attr, jnp.zeros((pad, D_E), jnp.float32)])

    q0, q1, q2 = _q_all(ea_p, We0, We1, We2)
    qs = [q.reshape(NW, NCHUNK, CHUNK, H) for q in (q0, q1, q2)]

    P, Rm = _pre0(x, Wm0, Wr0)
    layer = ((b0, g0, be0, Wm1, Wr1), (b1, g1, be1, Wm2, Wr2))
    for l in range(3):
        aggA, aggB = _sc_agg(P, qs[l], src_p, dst_p)
        if l < 2:
            bias, g, be, Wm, Wr = layer[l]
            P, Rm = _pre(Rm, aggA, aggB, bias, g, be, Wm, Wr)

    out = _readout(Rm, aggA, aggB, b2, g2, be2, batch,
                   RW1, Rb1, Rg, Rbe, RW2, Rb2)
    return out[:, 0]


# speedup vs baseline: 4.9134x; 1.0384x over previous
import functools

import jax
import jax.numpy as jnp
from jax import lax
from jax.experimental import pallas as pl
from jax.experimental.pallas import tpu as pltpu
from jax.experimental.pallas import tpu_sc as plsc

N = 10000
E = 320000
D_IN = 128
D_E = 16
H = 64
G = 64

NC = 2
NS = 16
NW = NC * NS
CHUNK = 64
NCHUNK = 160
NPAIR = NCHUNK // 2
E_PAD = NW * NCHUNK * CHUNK
ROWS_PER_TILE = N // NS
NEG = -1.0e30


RPW = NCHUNK * CHUNK


def _q_body(ea_ref, we0_ref, we1_ref, we2_ref, q0_ref, q1_ref, q2_ref):
    i = pl.program_id(0)
    ea = ea_ref[...]
    rows = i * RPW + lax.broadcasted_iota(jnp.int32, (RPW, 1), 0)
    valid = rows < E
    for we_ref, q_ref in ((we0_ref, q0_ref), (we1_ref, q1_ref), (we2_ref, q2_ref)):
        q = jnp.dot(ea, we_ref[...], preferred_element_type=jnp.float32)
        q_ref[...] = jnp.where(valid, q, NEG).reshape(1, NCHUNK, CHUNK, H)


def _q_all(ea, We0, We1, We2):
    return pl.pallas_call(
        _q_body,
        grid=(NW,),
        in_specs=[
            pl.BlockSpec((RPW, D_E), lambda i: (i, 0)),
            pl.BlockSpec((D_E, H), lambda i: (0, 0)),
            pl.BlockSpec((D_E, H), lambda i: (0, 0)),
            pl.BlockSpec((D_E, H), lambda i: (0, 0)),
        ],
        out_specs=[
            pl.BlockSpec((1, NCHUNK, CHUNK, H), lambda i: (i, 0, 0, 0)),
            pl.BlockSpec((1, NCHUNK, CHUNK, H), lambda i: (i, 0, 0, 0)),
            pl.BlockSpec((1, NCHUNK, CHUNK, H), lambda i: (i, 0, 0, 0)),
        ],
        out_shape=[jax.ShapeDtypeStruct((NW, NCHUNK, CHUNK, H), jnp.float32)] * 3,
    )(ea, We0, We1, We2)


def _pre0_body(x_ref, wm_ref, wr_ref, p_ref, r_ref):
    x = x_ref[...]
    p_ref[...] = jnp.dot(x, wm_ref[...], preferred_element_type=jnp.float32)
    r_ref[...] = jnp.dot(x, wr_ref[...], preferred_element_type=jnp.float32)


def _pre0(x, Wm, Wr):
    return pl.pallas_call(
        _pre0_body,
        out_shape=[jax.ShapeDtypeStruct((N, H), jnp.float32)] * 2,
    )(x, Wm, Wr)


def _pre_body(r_ref, a_ref, b_ref, bias_ref, g_ref, be_ref, wm_ref, wr_ref,
              p_out, r_out):
    t = r_ref[...] + a_ref[...] + b_ref[...] + bias_ref[...]
    mu = jnp.mean(t, axis=0, keepdims=True)
    var = jnp.mean((t - mu) ** 2, axis=0, keepdims=True)
    h = jnp.maximum(g_ref[...] * (t - mu) / jnp.sqrt(var + 1e-5) + be_ref[...], 0.0)
    p_out[...] = jnp.dot(h, wm_ref[...], preferred_element_type=jnp.float32)
    r_out[...] = jnp.dot(h, wr_ref[...], preferred_element_type=jnp.float32)


def _pre(Rm, aggA, aggB, bias, g, be, Wm, Wr):
    return pl.pallas_call(
        _pre_body,
        out_shape=[jax.ShapeDtypeStruct((N, H), jnp.float32)] * 2,
    )(Rm, aggA, aggB, bias.reshape(1, H), g.reshape(1, H), be.reshape(1, H),
      Wm, Wr)


def _readout_body(r_ref, a_ref, b_ref, bias_ref, g_ref, be_ref, batch_ref,
                  rw1_ref, rb1_ref, rg_ref, rbe_ref, rw2_ref, rb2_ref, out_ref):
    t = r_ref[...] + a_ref[...] + b_ref[...] + bias_ref[...]
    mu = jnp.mean(t, axis=0, keepdims=True)
    var = jnp.mean((t - mu) ** 2, axis=0, keepdims=True)
    h = jnp.maximum(g_ref[...] * (t - mu) / jnp.sqrt(var + 1e-5) + be_ref[...], 0.0)
    gid = lax.broadcasted_iota(jnp.int32, (G, N), 0)
    onehot = (gid == batch_ref[...]).astype(jnp.float32)
    s = jnp.dot(onehot, h, preferred_element_type=jnp.float32,
                precision=lax.Precision.HIGHEST)
    cnt = jnp.sum(onehot, axis=1, keepdims=True)
    emb = s / jnp.maximum(cnt, 1.0)
    z1 = jnp.dot(emb, rw1_ref[...], preferred_element_type=jnp.float32) + rb1_ref[...]
    mu2 = jnp.mean(z1, axis=0, keepdims=True)
    var2 = jnp.mean((z1 - mu2) ** 2, axis=0, keepdims=True)
    z = jnp.maximum(rg_ref[...] * (z1 - mu2) / jnp.sqrt(var2 + 1e-5) + rbe_ref[...], 0.0)
    out_ref[...] = jnp.dot(z, rw2_ref[...], preferred_element_type=jnp.float32) + rb2_ref[...]


def _readout(Rm, aggA, aggB, bias, g, be, batch, RW1, Rb1, Rg, Rbe, RW2, Rb2):
    return pl.pallas_call(
        _readout_body,
        out_shape=jax.ShapeDtypeStruct((G, 1), jnp.float32),
    )(Rm, aggA, aggB, bias.reshape(1, H), g.reshape(1, H), be.reshape(1, H),
      batch.reshape(1, N), RW1, Rb1.reshape(1, H // 2), Rg.reshape(1, H // 2),
      Rbe.reshape(1, H // 2), RW2, Rb2.reshape(1, 1))


def _sc_body(p_hbm, q_hbm, src_hbm, dst_hbm, outa_hbm, outb_hbm,
             src_v, dst_v, rows_a, rows_b, q_a, q_b, zbuf_v, agg_sh, p_sh,
             gsa, gsb, qsa, qsb):
    cid = lax.axis_index("c")
    sid = lax.axis_index("s")
    wid = sid * NC + cid

    @pl.when(sid < NS - 1)
    def _():
        st = pl.ds(sid * 624, 624)
        pltpu.sync_copy(p_hbm.at[st], p_sh.at[st])

    @pl.when(sid == NS - 1)
    def _():
        st = pl.ds(624 * (NS - 1), 640)
        pltpu.sync_copy(p_hbm.at[st], p_sh.at[st])

    def _zrow(r, carry):
        for j in range(H // 16):
            zbuf_v[r, pl.ds(16 * j, 16)] = jnp.zeros((16,), jnp.float32)
        return carry
    lax.fori_loop(0, 125, _zrow, 0)
    for k in range(5):
        pltpu.sync_copy(zbuf_v, agg_sh.at[pl.ds(sid * ROWS_PER_TILE + k * 125, 125)])
    plsc.subcore_barrier()

    pltpu.sync_copy(src_hbm.at[wid], src_v)
    pltpu.sync_copy(dst_hbm.at[wid], dst_v)

    def _relu_add(rows, q):
        def _row(e, inner):
            for j in range(H // 16):
                s = pl.ds(16 * j, 16)
                rows[e, s] = jnp.maximum(rows[e, s] + q[e, s], 0.0)
            return inner
        lax.fori_loop(0, CHUNK, _row, 0)

    def _start(c, rows, q, gs, qs):
        pltpu.async_copy(p_sh.at[src_v.at[c]], rows, gs)
        pltpu.async_copy(q_hbm.at[wid, c], q, qs)

    def _wait(c, rows, q, gs, qs):
        pltpu.make_async_copy(p_sh.at[src_v.at[c]], rows, gs).wait()
        pltpu.make_async_copy(q_hbm.at[wid, c], q, qs).wait()

    _start(0, rows_a, q_a, gsa, qsa)

    def _pair(c2, carry):
        c0 = 2 * c2
        c1 = c0 + 1
        _wait(c0, rows_a, q_a, gsa, qsa)
        _start(c1, rows_b, q_b, gsb, qsb)
        _relu_add(rows_a, q_a)
        pltpu.sync_copy(rows_a, agg_sh.at[dst_v.at[c0]], add=True)
        _wait(c1, rows_b, q_b, gsb, qsb)

        @pl.when(c2 + 1 < NPAIR)
        def _():
            _start(c0 + 2, rows_a, q_a, gsa, qsa)
        _relu_add(rows_b, q_b)
        pltpu.sync_copy(rows_b, agg_sh.at[dst_v.at[c1]], add=True)
        return carry
    lax.fori_loop(0, NPAIR, _pair, 0)
    plsc.subcore_barrier()

    def _drain(out):
        @pl.when(sid < NS - 1)
        def _():
            st = pl.ds(sid * 624, 624)
            pltpu.sync_copy(agg_sh.at[st], out.at[st])

        @pl.when(sid == NS - 1)
        def _():
            st = pl.ds(624 * (NS - 1), 640)
            pltpu.sync_copy(agg_sh.at[st], out.at[st])

    @pl.when(cid == 0)
    def _():
        _drain(outa_hbm)

    @pl.when(cid == 1)
    def _():
        _drain(outb_hbm)


_sc_agg = pl.kernel(
    _sc_body,
    out_type=[jax.ShapeDtypeStruct((N, H), jnp.float32)] * 2,
    mesh=plsc.VectorSubcoreMesh(core_axis_name="c", subcore_axis_name="s"),
    compiler_params=pltpu.CompilerParams(use_tc_tiling_on_sc=False),
    scratch_types=[
        pltpu.VMEM((NCHUNK, CHUNK), jnp.int32),
        pltpu.VMEM((NCHUNK, CHUNK), jnp.int32),
        pltpu.VMEM((CHUNK, H), jnp.float32),
        pltpu.VMEM((CHUNK, H), jnp.float32),
        pltpu.VMEM((CHUNK, H), jnp.float32),
        pltpu.VMEM((CHUNK, H), jnp.float32),
        pltpu.VMEM((125, H), jnp.float32),
        pltpu.VMEM_SHARED((N, H), jnp.float32),
        pltpu.VMEM_SHARED((N, H), jnp.float32),
        pltpu.SemaphoreType.DMA,
        pltpu.SemaphoreType.DMA,
        pltpu.SemaphoreType.DMA,
        pltpu.SemaphoreType.DMA,
    ],
)


def kernel(x, edge_index, edge_attr, batch,
           Wm0, Wr0, We0, b0, g0, be0,
           Wm1, Wr1, We1, b1, g1, be1,
           Wm2, Wr2, We2, b2, g2, be2,
           RW1, Rb1, Rg, Rbe, RW2, Rb2):
    pad = E_PAD - E
    src = jnp.concatenate([edge_index[0], jnp.zeros((pad,), jnp.int32)])
    dst = jnp.concatenate([edge_index[1], jnp.zeros((pad,), jnp.int32)])
    src_p = src.reshape(NW, NCHUNK, CHUNK)
    dst_p = dst.reshape(NW, NCHUNK, CHUNK)

    qs = _q_all(edge_attr, We0, We1, We2)

    P, Rm = _pre0(x, Wm0, Wr0)
    layer = ((b0, g0, be0, Wm1, Wr1), (b1, g1, be1, Wm2, Wr2))
    for l in range(3):
        aggA, aggB = _sc_agg(P, qs[l], src_p, dst_p)
        if l < 2:
            bias, g, be, Wm, Wr = layer[l]
            P, Rm = _pre(Rm, aggA, aggB, bias, g, be, Wm, Wr)

    out = _readout(Rm, aggA, aggB, b2, g2, be2, batch,
                   RW1, Rb1, Rg, Rbe, RW2, Rb2)
    return out[:, 0]
